# Initial kernel scaffold; baseline (speedup 1.0000x reference)
import jax
import jax.numpy as jnp
from jax.experimental import pallas as pl


def kernel(x, edge_index, batch, W, b, W_pool):
    raise NotImplementedError("write your pallas kernel here")


# speedup vs baseline: 8.9314x; 8.9314x over previous
import functools

import jax
import jax.numpy as jnp
from jax import lax
from jax.experimental import pallas as pl
from jax.experimental.pallas import tpu as pltpu
from jax.experimental.pallas import tpu_sc as plsc

_N = 10000
_E = 320000
_D = 128
_G = 16

_NC = 2
_NS = 16
_NW = _NC * _NS
_EPT = _E // _NW
_CHUNK = 80
_NCHUNK = _EPT // _CHUNK
_RB = 624
_RREM = _N - _NS * _RB

_sc_mesh = plsc.VectorSubcoreMesh(core_axis_name="c", subcore_axis_name="s")


@functools.partial(
    pl.kernel,
    mesh=_sc_mesh,
    out_type=(
        jax.ShapeDtypeStruct((_NC, _N, _D), jnp.float32),
        jax.ShapeDtypeStruct((_NW, _N), jnp.float32),
    ),
    scratch_types=(
        pltpu.VMEM((_NCHUNK, _CHUNK), jnp.int32),
        pltpu.VMEM((_NCHUNK, _CHUNK), jnp.int32),
        pltpu.VMEM((_CHUNK, _D), jnp.float32),
        pltpu.VMEM((_N,), jnp.float32),
        pltpu.VMEM_SHARED((_N, _D), jnp.float32),
        pltpu.SemaphoreType.DMA,
    ),
    compiler_params=pltpu.CompilerParams(
        needs_layout_passes=False, use_tc_tiling_on_sc=False),
)
def _sc_edge_agg(x_hbm, src_hbm, dst_hbm, zn_hbm, zd_hbm,
                 agg_out, deg_out,
                 src_v, dst_v, rows_v, deg_v, agg_sh, sem):
    c = lax.axis_index("c")
    s = lax.axis_index("s")
    wid = c * _NS + s

    pltpu.sync_copy(src_hbm.at[wid], src_v)
    pltpu.sync_copy(dst_hbm.at[wid], dst_v)
    pltpu.sync_copy(zd_hbm, deg_v)
    r0 = s * _RB
    pltpu.sync_copy(zn_hbm.at[pl.ds(r0, _RB)], agg_sh.at[pl.ds(r0, _RB)])

    @pl.when(s == _NS - 1)
    def _zero_rem():
        rr = _NS * _RB
        pltpu.sync_copy(zn_hbm.at[pl.ds(rr, _RREM)], agg_sh.at[pl.ds(rr, _RREM)])

    plsc.subcore_barrier()

    def body(j, carry):
        pltpu.async_copy(x_hbm.at[src_v.at[j]], rows_v, sem).wait()
        pltpu.sync_copy(rows_v, agg_sh.at[dst_v.at[j]], add=True)
        for k in range(_CHUNK // 16):
            idx16 = dst_v[j, pl.ds(k * 16, 16)]
            counts, lastm = plsc.scan_count(idx16)
            plsc.addupdate_scatter(
                deg_v, [idx16], counts.astype(jnp.float32), mask=lastm)
        return carry

    lax.fori_loop(0, _NCHUNK, body, 0)

    plsc.subcore_barrier()
    pltpu.sync_copy(agg_sh.at[pl.ds(r0, _RB)], agg_out.at[c, pl.ds(r0, _RB)])
    pltpu.sync_copy(deg_v, deg_out.at[wid])

    @pl.when(s == _NS - 1)
    def _drain_rem():
        rr = _NS * _RB
        pltpu.sync_copy(agg_sh.at[pl.ds(rr, _RREM)],
                        agg_out.at[c, pl.ds(rr, _RREM)])


def _tc_body(x_ref, agg_ref, deg_ref, batch_ref, w_ref, b_ref, wp_ref,
             out_ref):
    agg = agg_ref[0] + agg_ref[1]
    degc = lax.dot_general(
        deg_ref[...], jnp.full((_NW, 1), 1.0, jnp.float32),
        (((0,), (0,)), ((), ())), preferred_element_type=jnp.float32)
    aggn = agg / jnp.maximum(degc, 1.0)
    h = jnp.maximum(
        jnp.dot(x_ref[...] + aggn, w_ref[...],
                preferred_element_type=jnp.float32) + b_ref[...],
        0.0)
    seg = (lax.broadcasted_iota(jnp.int32, (_G, _N), 0)
           == batch_ref[...]).astype(jnp.float32)
    gsum = jnp.dot(seg, h, preferred_element_type=jnp.float32)
    gcnt = jnp.sum(seg, axis=1, keepdims=True)
    gmean = gsum / jnp.maximum(gcnt, 1.0)
    gfeat = jnp.dot(gmean, wp_ref[...], preferred_element_type=jnp.float32)
    out_ref[...] = h + lax.dot_general(
        seg, gfeat, (((0,), (0,)), ((), ())),
        preferred_element_type=jnp.float32)


@jax.jit
def _impl(x, src3, dst3, batch2, w, b2, wp):
    zn = jnp.zeros((_N, _D), jnp.float32)
    zd = jnp.zeros((_N,), jnp.float32)
    agg2, deg2 = _sc_edge_agg(x, src3, dst3, zn, zd)
    return pl.pallas_call(
        _tc_body,
        out_shape=jax.ShapeDtypeStruct((_N, _D), jnp.float32),
    )(x, agg2, deg2, batch2, w, b2, wp)


def kernel(x, edge_index, batch, W, b, W_pool):
    ei = edge_index.astype(jnp.int32)
    src3 = ei[0].reshape(_NW, _NCHUNK, _CHUNK)
    dst3 = ei[1].reshape(_NW, _NCHUNK, _CHUNK)
    batch2 = batch.astype(jnp.int32).reshape(1, _N)
    return _impl(x, src3, dst3, batch2, W, b.reshape(1, _D), W_pool)


# speedup vs baseline: 11.4839x; 1.2858x over previous
import functools

import jax
import jax.numpy as jnp
from jax import lax
from jax.experimental import pallas as pl
from jax.experimental.pallas import tpu as pltpu
from jax.experimental.pallas import tpu_sc as plsc

_N = 10000
_E = 320000
_D = 128
_G = 16

_NC = 2
_NS = 16
_NW = _NC * _NS
_EPT = _E // _NW
_CHUNK = 80
_NCHUNK = _EPT // _CHUNK
_RB = 624
_RREM = _N - _NS * _RB

_sc_mesh = plsc.VectorSubcoreMesh(core_axis_name="c", subcore_axis_name="s")


@functools.partial(
    pl.kernel,
    mesh=_sc_mesh,
    out_type=(
        jax.ShapeDtypeStruct((_NC, _N, _D), jnp.float32),
        jax.ShapeDtypeStruct((_NW, _N), jnp.float32),
    ),
    scratch_types=(
        pltpu.VMEM((_NCHUNK, _CHUNK), jnp.int32),
        pltpu.VMEM((_NCHUNK, _CHUNK), jnp.int32),
        pltpu.VMEM((_CHUNK, _D), jnp.float32),
        pltpu.VMEM((_CHUNK, _D), jnp.float32),
        pltpu.VMEM((_N,), jnp.float32),
        pltpu.VMEM_SHARED((_N, _D), jnp.float32),
        pltpu.SemaphoreType.DMA,
        pltpu.SemaphoreType.DMA,
    ),
    compiler_params=pltpu.CompilerParams(
        needs_layout_passes=False, use_tc_tiling_on_sc=False),
)
def _sc_edge_agg(x_hbm, src_hbm, dst_hbm, zn_hbm, zd_hbm,
                 agg_out, deg_out,
                 src_v, dst_v, rows_a, rows_b, deg_v, agg_sh, sem_a, sem_b):
    c = lax.axis_index("c")
    s = lax.axis_index("s")
    wid = c * _NS + s

    pltpu.sync_copy(src_hbm.at[wid], src_v)
    pltpu.sync_copy(dst_hbm.at[wid], dst_v)
    pltpu.sync_copy(zd_hbm, deg_v)
    r0 = s * _RB
    pltpu.sync_copy(zn_hbm.at[pl.ds(r0, _RB)], agg_sh.at[pl.ds(r0, _RB)])

    @pl.when(s == _NS - 1)
    def _zero_rem():
        rr = _NS * _RB
        pltpu.sync_copy(zn_hbm.at[pl.ds(rr, _RREM)], agg_sh.at[pl.ds(rr, _RREM)])

    plsc.subcore_barrier()

    def _gather(j, buf, sem):
        pltpu.async_copy(x_hbm.at[src_v.at[j]], buf, sem)

    def _gwait(buf, sem):
        pltpu.make_async_copy(x_hbm.at[pl.ds(0, _CHUNK)], buf, sem).wait()

    def _consume(j, buf):
        pltpu.sync_copy(buf, agg_sh.at[dst_v.at[j]], add=True)
        for k in range(_CHUNK // 16):
            idx16 = dst_v[j, pl.ds(k * 16, 16)]
            counts, lastm = plsc.scan_count(idx16)
            plsc.addupdate_scatter(
                deg_v, [idx16], counts.astype(jnp.float32), mask=lastm)

    _gather(0, rows_a, sem_a)

    def body(j2, carry):
        j = 2 * j2
        _gwait(rows_a, sem_a)
        _gather(j + 1, rows_b, sem_b)
        _consume(j, rows_a)
        _gwait(rows_b, sem_b)
        _gather(j + 2, rows_a, sem_a)
        _consume(j + 1, rows_b)
        return carry

    lax.fori_loop(0, (_NCHUNK - 1) // 2, body, 0)
    _gwait(rows_a, sem_a)
    _consume(_NCHUNK - 1, rows_a)

    plsc.subcore_barrier()
    pltpu.sync_copy(agg_sh.at[pl.ds(r0, _RB)], agg_out.at[c, pl.ds(r0, _RB)])
    pltpu.sync_copy(deg_v, deg_out.at[wid])

    @pl.when(s == _NS - 1)
    def _drain_rem():
        rr = _NS * _RB
        pltpu.sync_copy(agg_sh.at[pl.ds(rr, _RREM)],
                        agg_out.at[c, pl.ds(rr, _RREM)])


def _tc_body(x_ref, agg_ref, deg_ref, batch_ref, w_ref, b_ref, wp_ref,
             out_ref):
    agg = agg_ref[0] + agg_ref[1]
    degc = lax.dot_general(
        deg_ref[...], jnp.full((_NW, 1), 1.0, jnp.float32),
        (((0,), (0,)), ((), ())), preferred_element_type=jnp.float32)
    aggn = agg / jnp.maximum(degc, 1.0)
    h = jnp.maximum(
        jnp.dot(x_ref[...] + aggn, w_ref[...],
                preferred_element_type=jnp.float32) + b_ref[...],
        0.0)
    seg = (lax.broadcasted_iota(jnp.int32, (_G, _N), 0)
           == batch_ref[...]).astype(jnp.float32)
    gsum = jnp.dot(seg, h, preferred_element_type=jnp.float32)
    gcnt = jnp.sum(seg, axis=1, keepdims=True)
    gmean = gsum / jnp.maximum(gcnt, 1.0)
    gfeat = jnp.dot(gmean, wp_ref[...], preferred_element_type=jnp.float32)
    out_ref[...] = h + lax.dot_general(
        seg, gfeat, (((0,), (0,)), ((), ())),
        preferred_element_type=jnp.float32)


@jax.jit
def _impl(x, src3, dst3, batch2, w, b2, wp):
    zn = jnp.zeros((_N, _D), jnp.float32)
    zd = jnp.zeros((_N,), jnp.float32)
    agg2, deg2 = _sc_edge_agg(x, src3, dst3, zn, zd)
    return pl.pallas_call(
        _tc_body,
        out_shape=jax.ShapeDtypeStruct((_N, _D), jnp.float32),
    )(x, agg2, deg2, batch2, w, b2, wp)


def kernel(x, edge_index, batch, W, b, W_pool):
    ei = edge_index.astype(jnp.int32)
    src3 = ei[0].reshape(_NW, _NCHUNK, _CHUNK)
    dst3 = ei[1].reshape(_NW, _NCHUNK, _CHUNK)
    batch2 = batch.astype(jnp.int32).reshape(1, _N)
    return _impl(x, src3, dst3, batch2, W, b.reshape(1, _D), W_pool)


# speedup vs baseline: 14.0392x; 1.2225x over previous
import functools

import jax
import jax.numpy as jnp
from jax import lax
from jax.experimental import pallas as pl
from jax.experimental.pallas import tpu as pltpu
from jax.experimental.pallas import tpu_sc as plsc

_N = 10000
_E = 320000
_D = 128
_G = 16

_NC = 2
_NS = 16
_NW = _NC * _NS
_EPT = _E // _NW
_CHUNK = 80
_NCHUNK = _EPT // _CHUNK
_RB = 624
_RREM = _N - _NS * _RB

_sc_mesh = plsc.VectorSubcoreMesh(core_axis_name="c", subcore_axis_name="s")


@functools.partial(
    pl.kernel,
    mesh=_sc_mesh,
    out_type=(
        jax.ShapeDtypeStruct((_NC, _N, _D), jnp.float32),
        jax.ShapeDtypeStruct((_NW, _N), jnp.float32),
    ),
    scratch_types=(
        pltpu.VMEM((_NCHUNK, _CHUNK), jnp.int32),
        pltpu.VMEM((_NCHUNK, _CHUNK), jnp.int32),
        pltpu.VMEM((_CHUNK, _D), jnp.float32),
        pltpu.VMEM((_CHUNK, _D), jnp.float32),
        pltpu.VMEM((_N,), jnp.float32),
        pltpu.VMEM_SHARED((_N, _D), jnp.float32),
        pltpu.SemaphoreType.DMA,
        pltpu.SemaphoreType.DMA,
        pltpu.SemaphoreType.DMA,
        pltpu.SemaphoreType.DMA,
    ),
    compiler_params=pltpu.CompilerParams(
        needs_layout_passes=False, use_tc_tiling_on_sc=False),
)
def _sc_edge_agg(x_hbm, src_hbm, dst_hbm, zn_hbm, zd_hbm,
                 agg_out, deg_out,
                 src_v, dst_v, rows_a, rows_b, deg_v, agg_sh,
                 sem_a, sem_b, sem_sa, sem_sb):
    c = lax.axis_index("c")
    s = lax.axis_index("s")
    wid = c * _NS + s

    pltpu.sync_copy(src_hbm.at[wid], src_v)
    pltpu.sync_copy(dst_hbm.at[wid], dst_v)
    pltpu.sync_copy(zd_hbm, deg_v)
    r0 = s * _RB
    pltpu.sync_copy(zn_hbm.at[pl.ds(r0, _RB)], agg_sh.at[pl.ds(r0, _RB)])

    @pl.when(s == _NS - 1)
    def _zero_rem():
        rr = _NS * _RB
        pltpu.sync_copy(zn_hbm.at[pl.ds(rr, _RREM)], agg_sh.at[pl.ds(rr, _RREM)])

    plsc.subcore_barrier()

    def _gather(j, buf, sem):
        pltpu.async_copy(x_hbm.at[src_v.at[j]], buf, sem)

    def _gwait(buf, sem):
        pltpu.make_async_copy(x_hbm.at[pl.ds(0, _CHUNK)], buf, sem).wait()

    def _scat_start(j, buf, sem):
        pltpu.async_copy(buf, agg_sh.at[dst_v.at[j]], sem, add=True)

    def _scat_wait(j, buf, sem):
        pltpu.make_async_copy(buf, agg_sh.at[dst_v.at[j]], sem).wait()

    def _hist(j):
        for k in range(_CHUNK // 16):
            idx16 = dst_v[j, pl.ds(k * 16, 16)]
            counts, lastm = plsc.scan_count(idx16)
            plsc.addupdate_scatter(
                deg_v, [idx16], counts.astype(jnp.float32), mask=lastm)

    def _consume(j, buf, ssem):
        _scat_start(j, buf, ssem)
        _hist(j)
        _scat_wait(j, buf, ssem)

    _gather(0, rows_a, sem_a)
    _gather(1, rows_b, sem_b)

    def body(j2, carry):
        j = 2 * j2
        _gwait(rows_a, sem_a)
        _consume(j, rows_a, sem_sa)

        @pl.when(j + 2 < _NCHUNK)
        def _next_a():
            _gather(j + 2, rows_a, sem_a)

        _gwait(rows_b, sem_b)
        _consume(j + 1, rows_b, sem_sb)

        @pl.when(j + 3 < _NCHUNK)
        def _next_b():
            _gather(j + 3, rows_b, sem_b)

        return carry

    lax.fori_loop(0, _NCHUNK // 2, body, 0)
    if _NCHUNK % 2:
        _gwait(rows_a, sem_a)
        _consume(_NCHUNK - 1, rows_a, sem_sa)

    plsc.subcore_barrier()
    pltpu.sync_copy(agg_sh.at[pl.ds(r0, _RB)], agg_out.at[c, pl.ds(r0, _RB)])
    pltpu.sync_copy(deg_v, deg_out.at[wid])

    @pl.when(s == _NS - 1)
    def _drain_rem():
        rr = _NS * _RB
        pltpu.sync_copy(agg_sh.at[pl.ds(rr, _RREM)],
                        agg_out.at[c, pl.ds(rr, _RREM)])


def _tc_body(x_ref, agg_ref, deg_ref, batch_ref, w_ref, b_ref, wp_ref,
             out_ref):
    agg = agg_ref[0] + agg_ref[1]
    degc = lax.dot_general(
        deg_ref[...], jnp.full((_NW, 1), 1.0, jnp.float32),
        (((0,), (0,)), ((), ())), preferred_element_type=jnp.float32)
    aggn = agg / jnp.maximum(degc, 1.0)
    h = jnp.maximum(
        jnp.dot(x_ref[...] + aggn, w_ref[...],
                preferred_element_type=jnp.float32) + b_ref[...],
        0.0)
    seg = (lax.broadcasted_iota(jnp.int32, (_G, _N), 0)
           == batch_ref[...]).astype(jnp.float32)
    gsum = jnp.dot(seg, h, preferred_element_type=jnp.float32)
    gcnt = jnp.sum(seg, axis=1, keepdims=True)
    gmean = gsum / jnp.maximum(gcnt, 1.0)
    gfeat = jnp.dot(gmean, wp_ref[...], preferred_element_type=jnp.float32)
    out_ref[...] = h + lax.dot_general(
        seg, gfeat, (((0,), (0,)), ((), ())),
        preferred_element_type=jnp.float32)


@jax.jit
def _impl(x, src3, dst3, batch2, w, b2, wp):
    zn = jnp.zeros((_N, _D), jnp.float32)
    zd = jnp.zeros((_N,), jnp.float32)
    agg2, deg2 = _sc_edge_agg(x, src3, dst3, zn, zd)
    return pl.pallas_call(
        _tc_body,
        out_shape=jax.ShapeDtypeStruct((_N, _D), jnp.float32),
    )(x, agg2, deg2, batch2, w, b2, wp)


def kernel(x, edge_index, batch, W, b, W_pool):
    ei = edge_index.astype(jnp.int32)
    src3 = ei[0].reshape(_NW, _NCHUNK, _CHUNK)
    dst3 = ei[1].reshape(_NW, _NCHUNK, _CHUNK)
    batch2 = batch.astype(jnp.int32).reshape(1, _N)
    return _impl(x, src3, dst3, batch2, W, b.reshape(1, _D), W_pool)


# speedup vs baseline: 15.3221x; 1.0914x over previous
import functools

import jax
import jax.numpy as jnp
from jax import lax
from jax.experimental import pallas as pl
from jax.experimental.pallas import tpu as pltpu
from jax.experimental.pallas import tpu_sc as plsc

_N = 10000
_E = 320000
_D = 128
_G = 16

_NC = 2
_NS = 16
_NW = _NC * _NS
_EPT = _E // _NW
_CHUNK = 80
_NCHUNK = _EPT // _CHUNK
_RB = 624
_RREM = _N - _NS * _RB

_sc_mesh = plsc.VectorSubcoreMesh(core_axis_name="c", subcore_axis_name="s")


@functools.partial(
    pl.kernel,
    mesh=_sc_mesh,
    out_type=(
        jax.ShapeDtypeStruct((_NC, _N, _D), jnp.float32),
        jax.ShapeDtypeStruct((_NW, _N), jnp.float32),
    ),
    scratch_types=(
        [pltpu.VMEM((_CHUNK, _D), jnp.float32)] * 3,
        [pltpu.VMEM((_CHUNK,), jnp.int32)] * 3,
        [pltpu.VMEM((_CHUNK,), jnp.int32)] * 3,
        pltpu.VMEM((_N,), jnp.float32),
        pltpu.VMEM_SHARED((_N, _D), jnp.float32),
        [pltpu.SemaphoreType.DMA] * 3,
        [pltpu.SemaphoreType.DMA] * 3,
        [pltpu.SemaphoreType.DMA] * 3,
        [pltpu.SemaphoreType.DMA] * 3,
    ),
    compiler_params=pltpu.CompilerParams(
        needs_layout_passes=False, use_tc_tiling_on_sc=False),
)
def _sc_edge_agg(x_hbm, src_hbm, dst_hbm, zn_hbm, zd_hbm,
                 agg_out, deg_out,
                 rows, src_i, dst_i, deg_v, agg_sh,
                 gsem, ssem, issem, idsem):
    c = lax.axis_index("c")
    s = lax.axis_index("s")
    wid = c * _NS + s

    pltpu.sync_copy(zd_hbm, deg_v)
    r0 = s * _RB
    pltpu.sync_copy(zn_hbm.at[pl.ds(r0, _RB)], agg_sh.at[pl.ds(r0, _RB)])

    @pl.when(s == _NS - 1)
    def _zero_rem():
        rr = _NS * _RB
        pltpu.sync_copy(zn_hbm.at[pl.ds(rr, _RREM)], agg_sh.at[pl.ds(rr, _RREM)])

    plsc.subcore_barrier()

    def _load_src(j, k):
        pltpu.async_copy(src_hbm.at[wid, j], src_i[k], issem[k])

    def _load_dst(j, k):
        pltpu.async_copy(dst_hbm.at[wid, j], dst_i[k], idsem[k])

    def _wait_src(k):
        pltpu.make_async_copy(src_hbm.at[0, 0], src_i[k], issem[k]).wait()

    def _wait_dst(k):
        pltpu.make_async_copy(dst_hbm.at[0, 0], dst_i[k], idsem[k]).wait()

    def _gather(k):
        pltpu.async_copy(x_hbm.at[src_i[k]], rows[k], gsem[k])

    def _gwait(k):
        pltpu.make_async_copy(x_hbm.at[pl.ds(0, _CHUNK)], rows[k],
                              gsem[k]).wait()

    def _scat_start(k):
        pltpu.async_copy(rows[k], agg_sh.at[dst_i[k]], ssem[k], add=True)

    def _scat_wait(k):
        pltpu.make_async_copy(rows[k], agg_sh.at[dst_i[k]], ssem[k]).wait()

    def _hist(k):
        for g in range(_CHUNK // 16):
            idx16 = dst_i[k][pl.ds(g * 16, 16)]
            counts, lastm = plsc.scan_count(idx16)
            plsc.addupdate_scatter(
                deg_v, [idx16], counts.astype(jnp.float32), mask=lastm)

    def _step(j, k, guard):
        kq = (k + 2) % 3
        _gwait(k)
        _wait_dst(k)
        _scat_start(k)
        _hist(k)

        if guard:
            @pl.when(j >= 1)
            def _drain():
                _scat_wait(kq)
        else:
            _scat_wait(kq)

        _wait_src(kq)
        _gather(kq)
        _load_dst(j + 2, kq)

        if guard:
            @pl.when(j + 3 < _NCHUNK)
            def _pref():
                _load_src(j + 3, k)
        else:
            _load_src(j + 3, k)

    _load_src(0, 0)
    _load_src(1, 1)
    _load_src(2, 2)
    _load_dst(0, 0)
    _load_dst(1, 1)
    _wait_src(0)
    _gather(0)
    _wait_src(1)
    _gather(1)

    def body(q, carry):
        j = 3 * q
        _step(j, 0, True)
        _step(j + 1, 1, True)
        _step(j + 2, 2, True)
        return carry

    lax.fori_loop(0, (_NCHUNK - 2) // 3, body, 0)

    for j, k in ((_NCHUNK - 2, 0), (_NCHUNK - 1, 1)):
        kq = (k + 2) % 3
        _gwait(k)
        _wait_dst(k)
        _scat_start(k)
        _hist(k)
        _scat_wait(kq)
    _scat_wait(1)

    plsc.subcore_barrier()
    pltpu.sync_copy(agg_sh.at[pl.ds(r0, _RB)], agg_out.at[c, pl.ds(r0, _RB)])
    pltpu.sync_copy(deg_v, deg_out.at[wid])

    @pl.when(s == _NS - 1)
    def _drain_rem():
        rr = _NS * _RB
        pltpu.sync_copy(agg_sh.at[pl.ds(rr, _RREM)],
                        agg_out.at[c, pl.ds(rr, _RREM)])


def _tc_body(x_ref, agg_ref, deg_ref, batch_ref, w_ref, b_ref, wp_ref,
             out_ref):
    agg = agg_ref[0] + agg_ref[1]
    degc = lax.dot_general(
        deg_ref[...], jnp.full((_NW, 1), 1.0, jnp.float32),
        (((0,), (0,)), ((), ())), preferred_element_type=jnp.float32)
    aggn = agg / jnp.maximum(degc, 1.0)
    h = jnp.maximum(
        jnp.dot(x_ref[...] + aggn, w_ref[...],
                preferred_element_type=jnp.float32) + b_ref[...],
        0.0)
    seg = (lax.broadcasted_iota(jnp.int32, (_G, _N), 0)
           == batch_ref[...]).astype(jnp.float32)
    gsum = jnp.dot(seg, h, preferred_element_type=jnp.float32)
    gcnt = jnp.sum(seg, axis=1, keepdims=True)
    gmean = gsum / jnp.maximum(gcnt, 1.0)
    gfeat = jnp.dot(gmean, wp_ref[...], preferred_element_type=jnp.float32)
    out_ref[...] = h + lax.dot_general(
        seg, gfeat, (((0,), (0,)), ((), ())),
        preferred_element_type=jnp.float32)


@jax.jit
def _impl(x, src3, dst3, batch2, w, b2, wp):
    zn = jnp.zeros((_N, _D), jnp.float32)
    zd = jnp.zeros((_N,), jnp.float32)
    agg2, deg2 = _sc_edge_agg(x, src3, dst3, zn, zd)
    return pl.pallas_call(
        _tc_body,
        out_shape=jax.ShapeDtypeStruct((_N, _D), jnp.float32),
    )(x, agg2, deg2, batch2, w, b2, wp)


def kernel(x, edge_index, batch, W, b, W_pool):
    ei = edge_index.astype(jnp.int32)
    src3 = ei[0].reshape(_NW, _NCHUNK, _CHUNK)
    dst3 = ei[1].reshape(_NW, _NCHUNK, _CHUNK)
    batch2 = batch.astype(jnp.int32).reshape(1, _N)
    return _impl(x, src3, dst3, batch2, W, b.reshape(1, _D), W_pool)


# speedup vs baseline: 15.5235x; 1.0131x over previous
import functools

import jax
import jax.numpy as jnp
from jax import lax
from jax.experimental import pallas as pl
from jax.experimental.pallas import tpu as pltpu
from jax.experimental.pallas import tpu_sc as plsc

_N = 10000
_E = 320000
_D = 128
_G = 16

_NC = 2
_NS = 16
_NW = _NC * _NS
_EPT = _E // _NW
_CHUNK = 80
_NCHUNK = _EPT // _CHUNK
_RB = 624
_RREM = _N - _NS * _RB

_sc_mesh = plsc.VectorSubcoreMesh(core_axis_name="c", subcore_axis_name="s")


@functools.partial(
    pl.kernel,
    mesh=_sc_mesh,
    out_type=(
        jax.ShapeDtypeStruct((_NC, _N, _D), jnp.float32),
        jax.ShapeDtypeStruct((_NW, _N), jnp.float32),
    ),
    scratch_types=(
        [pltpu.VMEM((_CHUNK, _D), jnp.float32)] * 3,
        [pltpu.VMEM((_CHUNK,), jnp.int32)] * 3,
        [pltpu.VMEM((_CHUNK,), jnp.int32)] * 3,
        pltpu.VMEM((_N,), jnp.float32),
        pltpu.VMEM_SHARED((_N, _D), jnp.float32),
        [pltpu.SemaphoreType.DMA] * 3,
        [pltpu.SemaphoreType.DMA] * 3,
        [pltpu.SemaphoreType.DMA] * 3,
        [pltpu.SemaphoreType.DMA] * 3,
    ),
    compiler_params=pltpu.CompilerParams(
        needs_layout_passes=False, use_tc_tiling_on_sc=False),
)
def _sc_edge_agg(x_hbm, src_hbm, dst_hbm, zn_hbm, zd_hbm,
                 agg_out, deg_out,
                 rows, src_i, dst_i, deg_v, agg_sh,
                 gsem, ssem, issem, idsem):
    c = lax.axis_index("c")
    s = lax.axis_index("s")
    wid = c * _NS + s

    pltpu.sync_copy(zd_hbm, deg_v)
    r0 = s * _RB
    pltpu.sync_copy(zn_hbm, agg_sh.at[pl.ds(r0, _RB)])

    @pl.when(s == _NS - 1)
    def _zero_rem():
        rr = _NS * _RB
        pltpu.sync_copy(zn_hbm.at[pl.ds(0, _RREM)], agg_sh.at[pl.ds(rr, _RREM)])

    plsc.subcore_barrier()

    def _load_src(j, k):
        pltpu.async_copy(src_hbm.at[wid, j], src_i[k], issem[k])

    def _load_dst(j, k):
        pltpu.async_copy(dst_hbm.at[wid, j], dst_i[k], idsem[k])

    def _wait_src(k):
        pltpu.make_async_copy(src_hbm.at[0, 0], src_i[k], issem[k]).wait()

    def _wait_dst(k):
        pltpu.make_async_copy(dst_hbm.at[0, 0], dst_i[k], idsem[k]).wait()

    def _gather(k):
        pltpu.async_copy(x_hbm.at[src_i[k]], rows[k], gsem[k])

    def _gwait(k):
        pltpu.make_async_copy(x_hbm.at[pl.ds(0, _CHUNK)], rows[k],
                              gsem[k]).wait()

    def _scat_start(k):
        pltpu.async_copy(rows[k], agg_sh.at[dst_i[k]], ssem[k], add=True)

    def _scat_wait(k):
        pltpu.make_async_copy(rows[k], agg_sh.at[dst_i[k]], ssem[k]).wait()

    def _hist(k):
        for g in range(_CHUNK // 16):
            idx16 = dst_i[k][pl.ds(g * 16, 16)]
            counts, lastm = plsc.scan_count(idx16)
            plsc.addupdate_scatter(
                deg_v, [idx16], counts.astype(jnp.float32), mask=lastm)

    def _step(j, k, guard):
        kq = (k + 2) % 3
        _gwait(k)
        _wait_dst(k)
        _scat_start(k)
        _hist(k)

        if guard:
            @pl.when(j >= 1)
            def _drain():
                _scat_wait(kq)
        else:
            _scat_wait(kq)

        _wait_src(kq)
        _gather(kq)
        _load_dst(j + 2, kq)

        if guard:
            @pl.when(j + 3 < _NCHUNK)
            def _pref():
                _load_src(j + 3, k)
        else:
            _load_src(j + 3, k)

    _load_src(0, 0)
    _load_src(1, 1)
    _load_src(2, 2)
    _load_dst(0, 0)
    _load_dst(1, 1)
    _wait_src(0)
    _gather(0)
    _wait_src(1)
    _gather(1)

    def body(q, carry):
        j = 3 * q
        _step(j, 0, True)
        _step(j + 1, 1, True)
        _step(j + 2, 2, True)
        return carry

    lax.fori_loop(0, (_NCHUNK - 2) // 3, body, 0)

    for j, k in ((_NCHUNK - 2, 0), (_NCHUNK - 1, 1)):
        kq = (k + 2) % 3
        _gwait(k)
        _wait_dst(k)
        _scat_start(k)
        _hist(k)
        _scat_wait(kq)
    _scat_wait(1)

    plsc.subcore_barrier()
    pltpu.sync_copy(agg_sh.at[pl.ds(r0, _RB)], agg_out.at[c, pl.ds(r0, _RB)])
    pltpu.sync_copy(deg_v, deg_out.at[wid])

    @pl.when(s == _NS - 1)
    def _drain_rem():
        rr = _NS * _RB
        pltpu.sync_copy(agg_sh.at[pl.ds(rr, _RREM)],
                        agg_out.at[c, pl.ds(rr, _RREM)])


def _tc_body(x_ref, agg_ref, deg_ref, batch_ref, w_ref, b_ref, wp_ref,
             out_ref):
    agg = agg_ref[0] + agg_ref[1]
    degc = lax.dot_general(
        deg_ref[...], jnp.full((_NW, 1), 1.0, jnp.float32),
        (((0,), (0,)), ((), ())), preferred_element_type=jnp.float32)
    aggn = agg / jnp.maximum(degc, 1.0)
    h = jnp.maximum(
        jnp.dot(x_ref[...] + aggn, w_ref[...],
                preferred_element_type=jnp.float32) + b_ref[...],
        0.0)
    seg = (lax.broadcasted_iota(jnp.int32, (_G, _N), 0)
           == batch_ref[...]).astype(jnp.float32)
    gsum = jnp.dot(seg, h, preferred_element_type=jnp.float32)
    gcnt = jnp.sum(seg, axis=1, keepdims=True)
    gmean = gsum / jnp.maximum(gcnt, 1.0)
    gfeat = jnp.dot(gmean, wp_ref[...], preferred_element_type=jnp.float32)
    out_ref[...] = h + lax.dot_general(
        seg, gfeat, (((0,), (0,)), ((), ())),
        preferred_element_type=jnp.float32)


@jax.jit
def _impl(x, src3, dst3, batch2, w, b2, wp):
    zn = jnp.zeros((_RB, _D), jnp.float32)
    zd = jnp.zeros((_N,), jnp.float32)
    agg2, deg2 = _sc_edge_agg(x, src3, dst3, zn, zd)
    return pl.pallas_call(
        _tc_body,
        out_shape=jax.ShapeDtypeStruct((_N, _D), jnp.float32),
    )(x, agg2, deg2, batch2, w, b2, wp)


def kernel(x, edge_index, batch, W, b, W_pool):
    ei = edge_index.astype(jnp.int32)
    src3 = ei[0].reshape(_NW, _NCHUNK, _CHUNK)
    dst3 = ei[1].reshape(_NW, _NCHUNK, _CHUNK)
    batch2 = batch.astype(jnp.int32).reshape(1, _N)
    return _impl(x, src3, dst3, batch2, W, b.reshape(1, _D), W_pool)


# speedup vs baseline: 16.0935x; 1.0367x over previous
import functools

import jax
import jax.numpy as jnp
from jax import lax
from jax.experimental import pallas as pl
from jax.experimental.pallas import tpu as pltpu
from jax.experimental.pallas import tpu_sc as plsc

_N = 10000
_E = 320000
_D = 128
_G = 16

_NC = 2
_NS = 16
_NW = _NC * _NS
_EPT = _E // _NW
_CHUNK = 80
_NCHUNK = _EPT // _CHUNK
_RB = 624
_RREM = _N - _NS * _RB

_sc_mesh = plsc.VectorSubcoreMesh(core_axis_name="c", subcore_axis_name="s")


@functools.partial(
    pl.kernel,
    mesh=_sc_mesh,
    out_type=(
        jax.ShapeDtypeStruct((_NC, _N, _D), jnp.float32),
        jax.ShapeDtypeStruct((_NW, _N), jnp.float32),
    ),
    scratch_types=(
        [pltpu.VMEM((_CHUNK, _D), jnp.float32)] * 3,
        [pltpu.VMEM((_CHUNK,), jnp.int32)] * 3,
        [pltpu.VMEM((_CHUNK,), jnp.int32)] * 3,
        pltpu.VMEM((_N,), jnp.float32),
        pltpu.VMEM_SHARED((_N, _D), jnp.float32),
        [pltpu.SemaphoreType.DMA] * 3,
        [pltpu.SemaphoreType.DMA] * 3,
        [pltpu.SemaphoreType.DMA] * 3,
        [pltpu.SemaphoreType.DMA] * 3,
        [pltpu.SemaphoreType.DMA] * 2,
    ),
    compiler_params=pltpu.CompilerParams(
        needs_layout_passes=False, use_tc_tiling_on_sc=False),
)
def _sc_edge_agg(x_hbm, src_hbm, dst_hbm, zn_hbm, zd_hbm,
                 agg_out, deg_out,
                 rows, src_i, dst_i, deg_v, agg_sh,
                 gsem, ssem, issem, idsem, zsem):
    c = lax.axis_index("c")
    s = lax.axis_index("s")
    wid = c * _NS + s

    r0 = s * _RB
    zh1 = pltpu.async_copy(zd_hbm, deg_v, zsem[0])
    zh2 = pltpu.async_copy(zn_hbm, agg_sh.at[pl.ds(r0, _RB)], zsem[1])

    @pl.when(s == _NS - 1)
    def _zero_rem():
        rr = _NS * _RB
        pltpu.sync_copy(zn_hbm.at[pl.ds(0, _RREM)], agg_sh.at[pl.ds(rr, _RREM)])

    def _load_src(j, k):
        pltpu.async_copy(src_hbm.at[wid, j], src_i[k], issem[k])

    def _load_dst(j, k):
        pltpu.async_copy(dst_hbm.at[wid, j], dst_i[k], idsem[k])

    def _wait_src(k):
        pltpu.make_async_copy(src_hbm.at[0, 0], src_i[k], issem[k]).wait()

    def _wait_dst(k):
        pltpu.make_async_copy(dst_hbm.at[0, 0], dst_i[k], idsem[k]).wait()

    def _gather(k):
        pltpu.async_copy(x_hbm.at[src_i[k]], rows[k], gsem[k])

    def _gwait(k):
        pltpu.make_async_copy(x_hbm.at[pl.ds(0, _CHUNK)], rows[k],
                              gsem[k]).wait()

    def _scat_start(k):
        pltpu.async_copy(rows[k], agg_sh.at[dst_i[k]], ssem[k], add=True)

    def _scat_wait(k):
        pltpu.make_async_copy(rows[k], agg_sh.at[dst_i[k]], ssem[k]).wait()

    def _hist(k):
        for g in range(_CHUNK // 16):
            idx16 = dst_i[k][pl.ds(g * 16, 16)]
            counts, lastm = plsc.scan_count(idx16)
            plsc.addupdate_scatter(
                deg_v, [idx16], counts.astype(jnp.float32), mask=lastm)

    def _step(j, k, guard):
        kq = (k + 2) % 3
        _gwait(k)
        _wait_dst(k)
        _scat_start(k)
        _hist(k)

        if guard:
            @pl.when(j >= 1)
            def _drain():
                _scat_wait(kq)
        else:
            _scat_wait(kq)

        _wait_src(kq)
        _gather(kq)
        _load_dst(j + 2, kq)

        if guard:
            @pl.when(j + 3 < _NCHUNK)
            def _pref():
                _load_src(j + 3, k)
        else:
            _load_src(j + 3, k)

    _load_src(0, 0)
    _load_src(1, 1)
    _load_src(2, 2)
    _load_dst(0, 0)
    _load_dst(1, 1)
    _wait_src(0)
    _gather(0)
    _wait_src(1)
    _gather(1)
    zh1.wait()
    zh2.wait()
    plsc.subcore_barrier()

    def body(q, carry):
        j = 3 * q
        _step(j, 0, True)
        _step(j + 1, 1, True)
        _step(j + 2, 2, True)
        return carry

    lax.fori_loop(0, (_NCHUNK - 2) // 3, body, 0)

    for j, k in ((_NCHUNK - 2, 0), (_NCHUNK - 1, 1)):
        kq = (k + 2) % 3
        _gwait(k)
        _wait_dst(k)
        _scat_start(k)
        _hist(k)
        _scat_wait(kq)
    _scat_wait(1)

    plsc.subcore_barrier()
    dh1 = pltpu.async_copy(agg_sh.at[pl.ds(r0, _RB)],
                           agg_out.at[c, pl.ds(r0, _RB)], zsem[0])
    dh2 = pltpu.async_copy(deg_v, deg_out.at[wid], zsem[1])

    @pl.when(s == _NS - 1)
    def _drain_rem():
        rr = _NS * _RB
        pltpu.sync_copy(agg_sh.at[pl.ds(rr, _RREM)],
                        agg_out.at[c, pl.ds(rr, _RREM)])

    dh1.wait()
    dh2.wait()


def _tc_body(x_ref, agg_ref, deg_ref, batch_ref, w_ref, b_ref, wp_ref,
             out_ref):
    agg = agg_ref[0] + agg_ref[1]
    degc = lax.dot_general(
        deg_ref[...], jnp.full((_NW, 1), 1.0, jnp.float32),
        (((0,), (0,)), ((), ())), preferred_element_type=jnp.float32)
    aggn = agg / jnp.maximum(degc, 1.0)
    h = jnp.maximum(
        jnp.dot(x_ref[...] + aggn, w_ref[...],
                preferred_element_type=jnp.float32) + b_ref[...],
        0.0)
    seg = (lax.broadcasted_iota(jnp.int32, (_G, _N), 0)
           == batch_ref[...]).astype(jnp.float32)
    gsum = jnp.dot(seg, h, preferred_element_type=jnp.float32)
    gcnt = jnp.sum(seg, axis=1, keepdims=True)
    gmean = gsum / jnp.maximum(gcnt, 1.0)
    gfeat = jnp.dot(gmean, wp_ref[...], preferred_element_type=jnp.float32)
    out_ref[...] = h + lax.dot_general(
        seg, gfeat, (((0,), (0,)), ((), ())),
        preferred_element_type=jnp.float32)


@jax.jit
def _impl(x, src3, dst3, batch2, w, b2, wp):
    zn = jnp.zeros((_RB, _D), jnp.float32)
    zd = jnp.zeros((_N,), jnp.float32)
    agg2, deg2 = _sc_edge_agg(x, src3, dst3, zn, zd)
    return pl.pallas_call(
        _tc_body,
        out_shape=jax.ShapeDtypeStruct((_N, _D), jnp.float32),
    )(x, agg2, deg2, batch2, w, b2, wp)


def kernel(x, edge_index, batch, W, b, W_pool):
    ei = edge_index.astype(jnp.int32)
    src3 = ei[0].reshape(_NW, _NCHUNK, _CHUNK)
    dst3 = ei[1].reshape(_NW, _NCHUNK, _CHUNK)
    batch2 = batch.astype(jnp.int32).reshape(1, _N)
    return _impl(x, src3, dst3, batch2, W, b.reshape(1, _D), W_pool)


# speedup vs baseline: 16.3516x; 1.0160x over previous
import functools

import jax
import jax.numpy as jnp
from jax import lax
from jax.experimental import pallas as pl
from jax.experimental.pallas import tpu as pltpu
from jax.experimental.pallas import tpu_sc as plsc

_N = 10000
_E = 320000
_D = 128
_G = 16

_NC = 2
_NS = 16
_NW = _NC * _NS
_EPT = _E // _NW
_CHUNK = 80
_NCHUNK = _EPT // _CHUNK
_RB = 624
_RREM = _N - _NS * _RB

_sc_mesh = plsc.VectorSubcoreMesh(core_axis_name="c", subcore_axis_name="s")


@functools.partial(
    pl.kernel,
    mesh=_sc_mesh,
    out_type=(
        jax.ShapeDtypeStruct((_NC, _N, _D), jnp.float32),
        jax.ShapeDtypeStruct((_NW, _N), jnp.float32),
    ),
    scratch_types=(
        [pltpu.VMEM((_CHUNK, _D), jnp.float32)] * 3,
        [pltpu.VMEM((_CHUNK,), jnp.int32)] * 3,
        [pltpu.VMEM((_CHUNK,), jnp.int32)] * 3,
        pltpu.VMEM((_N,), jnp.float32),
        pltpu.VMEM_SHARED((_N, _D), jnp.float32),
        [pltpu.SemaphoreType.DMA] * 3,
        [pltpu.SemaphoreType.DMA] * 3,
        [pltpu.SemaphoreType.DMA] * 3,
        [pltpu.SemaphoreType.DMA] * 3,
        [pltpu.SemaphoreType.DMA] * 2,
    ),
    compiler_params=pltpu.CompilerParams(
        needs_layout_passes=False, use_tc_tiling_on_sc=False),
)
def _sc_edge_agg(x_hbm, src_hbm, dst_hbm, zn_hbm, zd_hbm,
                 agg_out, deg_out,
                 rows, src_i, dst_i, deg_v, agg_sh,
                 gsem, ssem, issem, idsem, zsem):
    c = lax.axis_index("c")
    s = lax.axis_index("s")
    wid = c * _NS + s

    r0 = s * _RB
    zh1 = pltpu.async_copy(zd_hbm, deg_v, zsem[0])
    zh2 = pltpu.async_copy(zn_hbm, agg_sh.at[pl.ds(r0, _RB)], zsem[1])

    @pl.when(s == _NS - 1)
    def _zero_rem():
        rr = _NS * _RB
        pltpu.sync_copy(zn_hbm.at[pl.ds(0, _RREM)], agg_sh.at[pl.ds(rr, _RREM)])

    def _load_src(j, k):
        pltpu.async_copy(src_hbm.at[wid, j], src_i[k], issem[k])

    def _load_dst(j, k):
        pltpu.async_copy(dst_hbm.at[wid, j], dst_i[k], idsem[k])

    def _wait_src(k):
        pltpu.make_async_copy(src_hbm.at[0, 0], src_i[k], issem[k]).wait()

    def _wait_dst(k):
        pltpu.make_async_copy(dst_hbm.at[0, 0], dst_i[k], idsem[k]).wait()

    def _gather(k):
        pltpu.async_copy(x_hbm.at[src_i[k]], rows[k], gsem[k])

    def _gwait(k):
        pltpu.make_async_copy(x_hbm.at[pl.ds(0, _CHUNK)], rows[k],
                              gsem[k]).wait()

    def _scat_start(k):
        pltpu.async_copy(rows[k], agg_sh.at[dst_i[k]], ssem[k], add=True)

    def _scat_wait(k):
        pltpu.make_async_copy(rows[k], agg_sh.at[dst_i[k]], ssem[k]).wait()

    def _hist(k):
        for g in range(_CHUNK // 16):
            idx16 = dst_i[k][pl.ds(g * 16, 16)]
            counts, lastm = plsc.scan_count(idx16)
            plsc.addupdate_scatter(
                deg_v, [idx16], counts.astype(jnp.float32), mask=lastm)

    def _step(j, k, guard):
        kq = (k + 2) % 3
        _gwait(k)
        _wait_dst(k)
        _scat_start(k)

        if guard:
            @pl.when(j >= 1)
            def _drain():
                _scat_wait(kq)
        else:
            _scat_wait(kq)

        _wait_src(kq)
        _gather(kq)
        _load_dst(j + 2, kq)

        if guard:
            @pl.when(j + 3 < _NCHUNK)
            def _pref():
                _load_src(j + 3, k)
        else:
            _load_src(j + 3, k)

        _hist(k)

    _load_src(0, 0)
    _load_src(1, 1)
    _load_src(2, 2)
    _load_dst(0, 0)
    _load_dst(1, 1)
    _wait_src(0)
    _gather(0)
    _wait_src(1)
    _gather(1)
    zh1.wait()
    zh2.wait()
    plsc.subcore_barrier()

    def body(q, carry):
        j = 3 * q
        _step(j, 0, True)
        _step(j + 1, 1, True)
        _step(j + 2, 2, True)
        return carry

    lax.fori_loop(0, (_NCHUNK - 2) // 3, body, 0)

    for j, k in ((_NCHUNK - 2, 0), (_NCHUNK - 1, 1)):
        kq = (k + 2) % 3
        _gwait(k)
        _wait_dst(k)
        _scat_start(k)
        _hist(k)
        _scat_wait(kq)
    _scat_wait(1)

    plsc.subcore_barrier()
    dh1 = pltpu.async_copy(agg_sh.at[pl.ds(r0, _RB)],
                           agg_out.at[c, pl.ds(r0, _RB)], zsem[0])
    dh2 = pltpu.async_copy(deg_v, deg_out.at[wid], zsem[1])

    @pl.when(s == _NS - 1)
    def _drain_rem():
        rr = _NS * _RB
        pltpu.sync_copy(agg_sh.at[pl.ds(rr, _RREM)],
                        agg_out.at[c, pl.ds(rr, _RREM)])

    dh1.wait()
    dh2.wait()


def _tc_body(x_ref, agg_ref, deg_ref, batch_ref, w_ref, b_ref, wp_ref,
             out_ref):
    agg = agg_ref[0] + agg_ref[1]
    degc = lax.dot_general(
        deg_ref[...], jnp.full((_NW, 1), 1.0, jnp.float32),
        (((0,), (0,)), ((), ())), preferred_element_type=jnp.float32)
    aggn = agg / jnp.maximum(degc, 1.0)
    h = jnp.maximum(
        jnp.dot(x_ref[...] + aggn, w_ref[...],
                preferred_element_type=jnp.float32) + b_ref[...],
        0.0)
    seg = (lax.broadcasted_iota(jnp.int32, (_G, _N), 0)
           == batch_ref[...]).astype(jnp.float32)
    gsum = jnp.dot(seg, h, preferred_element_type=jnp.float32)
    gcnt = jnp.sum(seg, axis=1, keepdims=True)
    gmean = gsum / jnp.maximum(gcnt, 1.0)
    gfeat = jnp.dot(gmean, wp_ref[...], preferred_element_type=jnp.float32)
    out_ref[...] = h + lax.dot_general(
        seg, gfeat, (((0,), (0,)), ((), ())),
        preferred_element_type=jnp.float32)


@jax.jit
def _impl(x, src3, dst3, batch2, w, b2, wp):
    zn = jnp.zeros((_RB, _D), jnp.float32)
    zd = jnp.zeros((_N,), jnp.float32)
    agg2, deg2 = _sc_edge_agg(x, src3, dst3, zn, zd)
    return pl.pallas_call(
        _tc_body,
        out_shape=jax.ShapeDtypeStruct((_N, _D), jnp.float32),
    )(x, agg2, deg2, batch2, w, b2, wp)


def kernel(x, edge_index, batch, W, b, W_pool):
    ei = edge_index.astype(jnp.int32)
    src3 = ei[0].reshape(_NW, _NCHUNK, _CHUNK)
    dst3 = ei[1].reshape(_NW, _NCHUNK, _CHUNK)
    batch2 = batch.astype(jnp.int32).reshape(1, _N)
    return _impl(x, src3, dst3, batch2, W, b.reshape(1, _D), W_pool)
